# baseline (speedup 1.0000x reference)
import functools

import jax
import jax.numpy as jnp
from jax import lax
from jax.experimental import pallas as pl
from jax.experimental.pallas import tpu as pltpu
from jax.experimental.pallas import tpu_sc as plsc

NUM_CORES = 2
NUM_SUBCORES = 16
NW = NUM_CORES * NUM_SUBCORES

BATCH = 16384
HIST = 50
HISTP = 128
DIM = 64
TOTAL = BATCH * HIST
PER_W = TOTAL // NW
B_PER_W = BATCH // NW
HISTA = 56
BCHUNK = 8
CHUNK = BCHUNK * HIST
CHUNKA = BCHUNK * HISTA
NCHUNK = B_PER_W // BCHUNK
NBUF = 2
NOUT = NCHUNK // NBUF

_mesh = plsc.VectorSubcoreMesh(
    core_axis_name="c", subcore_axis_name="s",
    num_cores=NUM_CORES, num_subcores=NUM_SUBCORES,
)


@functools.partial(
    pl.kernel,
    out_type=jax.ShapeDtypeStruct((NW, NCHUNK, CHUNK, DIM), jnp.float32),
    mesh=_mesh,
    scratch_types=[
        pltpu.VMEM((B_PER_W, HISTP), jnp.int32),
        *[pltpu.VMEM((CHUNKA, DIM), jnp.float32) for _ in range(NBUF)],
        *[pltpu.SemaphoreType.DMA for _ in range(NBUF)],
        *[pltpu.SemaphoreType.DMA for _ in range(NBUF)],
    ],
    compiler_params=pltpu.CompilerParams(use_tc_tiling_on_sc=False),
)
def _gather_sc(idx_hbm, table_hbm, out_hbm, idx_v, *scratch):
    bufs = scratch[:NBUF]
    gsem = scratch[NBUF:2 * NBUF]
    osem = scratch[2 * NBUF:]

    wid = lax.axis_index("s") * NUM_CORES + lax.axis_index("c")
    pltpu.sync_copy(idx_hbm.at[pl.ds(wid * B_PER_W, B_PER_W)], idx_v)

    def fire_gather(j, buf, sem):
        for k in range(BCHUNK):
            pltpu.async_copy(
                table_hbm.at[idx_v.at[j * BCHUNK + k, pl.ds(0, HISTA)]],
                buf.at[pl.ds(k * HISTA, HISTA)], sem)

    def wait_gather(buf, sem):
        for k in range(BCHUNK):
            pltpu.make_async_copy(
                table_hbm.at[pl.ds(0, HISTA)],
                buf.at[pl.ds(0, HISTA)], sem).wait()

    def fire_writeback(j, buf, sem):
        for k in range(BCHUNK):
            pltpu.async_copy(
                buf.at[pl.ds(k * HISTA, HIST)],
                out_hbm.at[wid, j, pl.ds(k * HIST, HIST)], sem)

    def wait_writeback(buf, sem):
        for k in range(BCHUNK):
            pltpu.make_async_copy(
                buf.at[pl.ds(0, HIST)],
                out_hbm.at[wid, 0, pl.ds(0, HIST)], sem).wait()

    for b in range(NBUF):
        fire_gather(b, bufs[b], gsem[b])

    def body(t, carry):
        j0 = t * NBUF
        for b in range(NBUF):
            j = j0 + b
            wait_gather(bufs[b], gsem[b])
            fire_writeback(j, bufs[b], osem[b])

            @pl.when(j + NBUF < NCHUNK)
            def _():
                wait_writeback(bufs[b], osem[b])
                fire_gather(j + NBUF, bufs[b], gsem[b])
        return carry

    lax.fori_loop(0, NOUT, body, 0)
    for b in range(NBUF):
        wait_writeback(bufs[b], osem[b])


def kernel(input_ids, weight):
    idx = jnp.pad(input_ids.astype(jnp.int32), ((0, 0), (0, HISTP - HIST)))
    out = _gather_sc(idx, weight)
    return out.reshape(BATCH, HIST, DIM)
